# baseline (speedup 1.0000x reference)
import functools

import jax
import jax.numpy as jnp
from jax import lax
from jax.experimental import pallas as pl
from jax.experimental.pallas import tpu as pltpu
from jax.experimental.pallas import tpu_sc as plsc

EPS = 1e-12
L = 16


def _rsqrt_scalar(x):
    i = lax.bitcast_convert_type(x, jnp.int32)
    y = lax.bitcast_convert_type(jnp.int32(0x5F3759DF) - (i >> 1), jnp.float32)
    half_x = jnp.float32(0.5) * x
    for _ in range(2):
        y = y * (jnp.float32(1.5) - half_x * y * y)
    return y


def _tree_sum(vs):
    while len(vs) > 1:
        vs = [a + b for a, b in zip(vs[::2], vs[1::2])]
    return vs[0]


@functools.lru_cache(maxsize=None)
def _build_sc_kernel(batch, seq, vocab, hid, n_workers):
    n_tok = batch * seq
    ppw = seq // n_workers
    ch = 256
    gch = 128
    gpc = ch // gch
    cpp = ppw // ch
    n_chunks = batch * cpp
    kf = hid // L

    mesh = plsc.VectorSubcoreMesh(core_axis_name="c", subcore_axis_name="s")
    nc = 2

    @functools.partial(
        pl.kernel,
        mesh=mesh,
        compiler_params=pltpu.CompilerParams(needs_layout_passes=False),
        out_type=jax.ShapeDtypeStruct((n_tok, hid), jnp.float32),
        scratch_types=[
            [pltpu.VMEM((gch,), jnp.int32)] * (n_chunks * gpc),
            [pltpu.VMEM((ch, hid), jnp.float32)] * 2,
            pltpu.VMEM((ppw, hid), jnp.float32),
            pltpu.VMEM((hid,), jnp.float32),
            pltpu.VMEM((hid,), jnp.float32),
            [pltpu.SemaphoreType.DMA] * 2,
            pltpu.SemaphoreType.DMA,
            pltpu.SemaphoreType.DMA,
            [pltpu.SemaphoreType.DMA] * 2,
        ],
    )
    def sc_kernel(ids_hbm, tab_hbm, pos_hbm, gam_hbm, bet_hbm, out_hbm,
                  idx_v, rows_v, pos_v, gam_v, bet_v, sem_g, sem_p, sem_i,
                  sem_o):
        wid = lax.axis_index("s") * nc + lax.axis_index("c")
        pbase = wid * ppw

        def tok_base(cidx):
            bseg, j = divmod(cidx, cpp)
            return bseg * seq + pbase + j * ch, j * ch

        id_cps = [
            pltpu.async_copy(
                ids_hbm.at[pl.ds(tok_base(g // gpc)[0] + (g % gpc) * gch, gch)],
                idx_v[g], sem_i)
            for g in range(n_chunks * gpc)
        ]
        pos_cp = pltpu.async_copy(pos_hbm.at[pl.ds(pbase, ppw)], pos_v, sem_p)
        gam_cp = pltpu.async_copy(gam_hbm, gam_v, sem_p)
        bet_cp = pltpu.async_copy(bet_hbm, bet_v, sem_p)

        inv_h = jnp.float32(1.0 / hid)

        def start_fetch(cidx, b):
            return [
                pltpu.async_copy(
                    tab_hbm.at[idx_v[cidx * gpc + g]],
                    rows_v[b].at[pl.ds(g * gch, gch)], sem_g[b])
                for g in range(gpc)
            ]

        for cp in id_cps:
            cp.wait()
        out_cp = [None, None]
        fetch = {0: start_fetch(0, 0)}
        gam_cp.wait()
        bet_cp.wait()
        pos_cp.wait()
        gam = [gam_v[pl.ds(k * L, L)] for k in range(kf)]
        bet = [bet_v[pl.ds(k * L, L)] for k in range(kf)]
        for cidx in range(n_chunks):
            b = cidx & 1
            if cidx + 1 < n_chunks:
                nb = (cidx + 1) & 1
                if out_cp[nb] is not None:
                    out_cp[nb].wait()
                    out_cp[nb] = None
                fetch[cidx + 1] = start_fetch(cidx + 1, nb)
            for cp in fetch.pop(cidx):
                cp.wait()
            t0, poff = tok_base(cidx)
            rv = rows_v[b]

            def tok_body(t):
                tp = t + poff
                e = [rv[t, pl.ds(k * L, L)] + pos_v[tp, pl.ds(k * L, L)]
                     for k in range(kf)]
                ssum = jnp.sum(_tree_sum(e))
                q = e[0] * e[0]
                for v in e[1:]:
                    q = q + v * v
                sqsum = jnp.sum(q)
                mean = ssum * inv_h
                var = sqsum * inv_h - mean * mean
                rstd = jnp.full((L,), _rsqrt_scalar(var + EPS), jnp.float32)
                mv = jnp.full((L,), mean, jnp.float32)
                a = [rstd * g for g in gam]
                for k in range(kf):
                    rv[t, pl.ds(k * L, L)] = (e[k] - mv) * a[k] + bet[k]

            plsc.parallel_loop(0, ch, unroll=4)(tok_body)
            out_cp[b] = pltpu.async_copy(
                rv, out_hbm.at[pl.ds(t0, ch)], sem_o[b])
        for cp in out_cp:
            if cp is not None:
                cp.wait()

    return sc_kernel


def kernel(input_ids, word_embeddings, position_embeddings, ln_gamma, ln_beta):
    batch, seq = input_ids.shape
    vocab, hid = word_embeddings.shape
    sc_kernel = _build_sc_kernel(batch, seq, vocab, hid, 32)
    out = sc_kernel(input_ids.reshape(-1), word_embeddings,
                    position_embeddings, ln_gamma, ln_beta)
    return out.reshape(batch, seq, hid)


# speedup vs baseline: 1.4349x; 1.4349x over previous
import functools

import jax
import jax.numpy as jnp
from jax import lax
from jax.experimental import pallas as pl
from jax.experimental.pallas import tpu as pltpu
from jax.experimental.pallas import tpu_sc as plsc

EPS = 1e-12
L = 16


def _rsqrt_scalar(x):
    i = lax.bitcast_convert_type(x, jnp.int32)
    y = lax.bitcast_convert_type(jnp.int32(0x5F3759DF) - (i >> 1), jnp.float32)
    half_x = jnp.float32(0.5) * x
    for _ in range(2):
        y = y * (jnp.float32(1.5) - half_x * y * y)
    return y


def _tree_sum(vs):
    while len(vs) > 1:
        vs = [a + b for a, b in zip(vs[::2], vs[1::2])]
    return vs[0]


@functools.lru_cache(maxsize=None)
def _build_sc_kernel(batch, seq, vocab, hid, n_workers):
    n_tok = batch * seq
    ppw = seq // n_workers
    ch = 256
    gch = 128
    gpc = ch // gch
    cpp = ppw // ch
    n_chunks = batch * cpp
    kf = hid // L

    mesh = plsc.VectorSubcoreMesh(core_axis_name="c", subcore_axis_name="s")
    nc = 2

    @functools.partial(
        pl.kernel,
        mesh=mesh,
        compiler_params=pltpu.CompilerParams(needs_layout_passes=False),
        out_type=jax.ShapeDtypeStruct((n_tok, hid), jnp.float32),
        scratch_types=[
            [pltpu.VMEM((gch,), jnp.int32)] * (n_chunks * gpc),
            [pltpu.VMEM((ch, hid), jnp.float32)] * 2,
            pltpu.VMEM((ppw, hid), jnp.float32),
            pltpu.VMEM((hid,), jnp.float32),
            pltpu.VMEM((hid,), jnp.float32),
            [pltpu.SemaphoreType.DMA] * 2,
            pltpu.SemaphoreType.DMA,
            pltpu.SemaphoreType.DMA,
            [pltpu.SemaphoreType.DMA] * 2,
        ],
    )
    def sc_kernel(ids_hbm, tab_hbm, pos_hbm, gam_hbm, bet_hbm, out_hbm,
                  idx_v, rows_v, pos_v, gam_v, bet_v, sem_g, sem_p, sem_i,
                  sem_o):
        wid = lax.axis_index("s") * nc + lax.axis_index("c")
        pbase = wid * ppw

        def tok_base(cidx):
            bseg, j = divmod(cidx, cpp)
            return bseg * seq + pbase + j * ch, j * ch

        id_cps = [
            pltpu.async_copy(
                ids_hbm.at[pl.ds(tok_base(g // gpc)[0] + (g % gpc) * gch, gch)],
                idx_v[g], sem_i)
            for g in range(n_chunks * gpc)
        ]
        pos_cp = pltpu.async_copy(pos_hbm.at[pl.ds(pbase, ppw)], pos_v, sem_p)
        gam_cp = pltpu.async_copy(gam_hbm, gam_v, sem_p)
        bet_cp = pltpu.async_copy(bet_hbm, bet_v, sem_p)

        inv_h = jnp.float32(1.0 / hid)

        def start_fetch(cidx, b):
            return [
                pltpu.async_copy(
                    tab_hbm.at[idx_v[cidx * gpc + g]],
                    rows_v[b].at[pl.ds(g * gch, gch)], sem_g[b])
                for g in range(gpc)
            ]

        for cp in id_cps:
            cp.wait()
        out_cp = [None, None]
        fetch = {0: start_fetch(0, 0)}
        gam_cp.wait()
        bet_cp.wait()
        pos_cp.wait()
        gam = [gam_v[pl.ds(k * L, L)] for k in range(kf)]
        bet = [bet_v[pl.ds(k * L, L)] for k in range(kf)]
        for cidx in range(n_chunks):
            b = cidx & 1
            if cidx + 1 < n_chunks:
                nb = (cidx + 1) & 1
                if out_cp[nb] is not None:
                    out_cp[nb].wait()
                    out_cp[nb] = None
                fetch[cidx + 1] = start_fetch(cidx + 1, nb)
            for cp in fetch.pop(cidx):
                cp.wait()
            t0, poff = tok_base(cidx)
            rv = rows_v[b]

            def tok_body(t):
                tp = t + poff
                e = [rv[t, pl.ds(k * L, L)] + pos_v[tp, pl.ds(k * L, L)]
                     for k in range(kf)]
                ssum = jnp.sum(_tree_sum(e))
                q = e[0] * e[0]
                for v in e[1:]:
                    q = q + v * v
                sqsum = jnp.sum(q)
                mean = ssum * inv_h
                var = sqsum * inv_h - mean * mean
                rstd = jnp.full((L,), _rsqrt_scalar(var + EPS), jnp.float32)
                mv = jnp.full((L,), mean, jnp.float32)
                a = [rstd * g for g in gam]
                for k in range(kf):
                    rv[t, pl.ds(k * L, L)] = (e[k] - mv) * a[k] + bet[k]

            plsc.parallel_loop(0, ch, unroll=2)(tok_body)
            out_cp[b] = pltpu.async_copy(
                rv, out_hbm.at[pl.ds(t0, ch)], sem_o[b])
        for cp in out_cp:
            if cp is not None:
                cp.wait()

    return sc_kernel


def kernel(input_ids, word_embeddings, position_embeddings, ln_gamma, ln_beta):
    batch, seq = input_ids.shape
    vocab, hid = word_embeddings.shape
    sc_kernel = _build_sc_kernel(batch, seq, vocab, hid, 32)
    out = sc_kernel(input_ids.reshape(-1), word_embeddings,
                    position_embeddings, ln_gamma, ln_beta)
    return out.reshape(batch, seq, hid)
